# initial kernel scaffold (unmeasured)
import jax
import jax.numpy as jnp
from jax import lax
from jax.experimental import pallas as pl
from jax.experimental.pallas import tpu as pltpu

N_DEV = 4
M, K_SHARD, N = 4096, 1024, 2048
M_CHUNK = M // N_DEV


def kernel(x, w_mat):
    x = x.astype(jnp.bfloat16)
    w = w_mat.astype(jnp.bfloat16)

    def body(x_ref, w_ref, out_ref, send_buf, recv_buf,
             send_sems, recv_sems, credit_sem):
        my = lax.axis_index("i")
        left = jnp.mod(my + N_DEV - 1, N_DEV)
        right = jnp.mod(my + 1, N_DEV)

        barrier_sem = pltpu.get_barrier_semaphore()
        for nbr in (left, right):
            pl.semaphore_signal(
                barrier_sem, inc=1,
                device_id=(nbr,), device_id_type=pl.DeviceIdType.MESH,
            )
        pl.semaphore_wait(barrier_sem, 2)

        out_ref[:, :] = jnp.dot(
            x_ref[:, :], w_ref[:, :], preferred_element_type=jnp.float32
        )

        def chunk(ref, c):
            return ref[pl.ds(c * M_CHUNK, M_CHUNK), :]

        def set_chunk(ref, c, val):
            ref[pl.ds(c * M_CHUNK, M_CHUNK), :] = val

        send_buf[:, :] = chunk(out_ref, jnp.mod(my + 3, N_DEV)).astype(
            jnp.bfloat16
        )

        for h in range(2 * (N_DEV - 1)):
            slot = h % 2
            if h >= 2:
                pl.semaphore_wait(credit_sem, 1)
            rdma = pltpu.make_async_remote_copy(
                src_ref=send_buf,
                dst_ref=recv_buf.at[slot],
                send_sem=send_sems.at[slot],
                recv_sem=recv_sems.at[slot],
                device_id=(right,),
                device_id_type=pl.DeviceIdType.MESH,
            )
            rdma.start()
            rdma.wait()

            recv_bf16 = recv_buf[slot, :, :]
            if h < N_DEV - 1:
                c_recv = jnp.mod(my + 2 - h + N_DEV, N_DEV)
                combined = chunk(out_ref, c_recv) + recv_bf16.astype(
                    jnp.float32
                )
                set_chunk(out_ref, c_recv, combined)
                if h == N_DEV - 2:
                    send_buf[:, :] = combined.astype(jnp.bfloat16)
                else:
                    send_buf[:, :] = combined.astype(jnp.bfloat16)
            else:
                g = h - (N_DEV - 1)
                c_recv = jnp.mod(my + N_DEV - 1 - g, N_DEV)
                set_chunk(out_ref, c_recv, recv_bf16.astype(jnp.float32))
                if h < 2 * (N_DEV - 1) - 1:
                    send_buf[:, :] = recv_bf16
            if h <= 3:
                pl.semaphore_signal(
                    credit_sem, inc=1,
                    device_id=(left,), device_id_type=pl.DeviceIdType.MESH,
                )

    return pl.pallas_call(
        body,
        out_shape=jax.ShapeDtypeStruct((M, N), jnp.float32),
        in_specs=[
            pl.BlockSpec(memory_space=pltpu.VMEM),
            pl.BlockSpec(memory_space=pltpu.VMEM),
        ],
        out_specs=pl.BlockSpec(memory_space=pltpu.VMEM),
        scratch_shapes=[
            pltpu.VMEM((M_CHUNK, N), jnp.bfloat16),
            pltpu.VMEM((2, M_CHUNK, N), jnp.bfloat16),
            pltpu.SemaphoreType.DMA((2,)),
            pltpu.SemaphoreType.DMA((2,)),
            pltpu.SemaphoreType.REGULAR,
        ],
        compiler_params=pltpu.CompilerParams(collective_id=0),
    )(x, w)


# baseline (device time: 360390 ns/iter reference)
import jax
import jax.numpy as jnp
from jax import lax
from jax.experimental import pallas as pl
from jax.experimental.pallas import tpu as pltpu

N_DEV = 4
M, K_SHARD, N = 4096, 1024, 2048
M_CHUNK = M // N_DEV


def kernel(x, w_mat):
    x = x.astype(jnp.bfloat16)
    w = w_mat.astype(jnp.bfloat16)

    def body(x_ref, w_ref, out_ref, send_buf, recv_buf,
             send_sems, recv_sems, credit_sem):
        my = lax.axis_index("i")
        left = jnp.mod(my + N_DEV - 1, N_DEV)
        right = jnp.mod(my + 1, N_DEV)

        barrier_sem = pltpu.get_barrier_semaphore()
        for nbr in (left, right):
            pl.semaphore_signal(
                barrier_sem, inc=1,
                device_id=(nbr,), device_id_type=pl.DeviceIdType.MESH,
            )
        pl.semaphore_wait(barrier_sem, 2)

        for c in range(N_DEV):
            out_ref[pl.ds(c * M_CHUNK, M_CHUNK), :] = jnp.dot(
                x_ref[pl.ds(c * M_CHUNK, M_CHUNK), :],
                w_ref[:, :],
                preferred_element_type=jnp.float32,
            )

        def chunk(ref, c):
            return ref[pl.ds(c * M_CHUNK, M_CHUNK), :]

        def set_chunk(ref, c, val):
            ref[pl.ds(c * M_CHUNK, M_CHUNK), :] = val

        send_buf[:, :] = chunk(out_ref, jnp.mod(my + 3, N_DEV)).astype(
            jnp.bfloat16
        )

        for h in range(2 * (N_DEV - 1)):
            slot = h % 2
            if h >= 2:
                pl.semaphore_wait(credit_sem, 1)
            rdma = pltpu.make_async_remote_copy(
                src_ref=send_buf,
                dst_ref=recv_buf.at[slot],
                send_sem=send_sems.at[slot],
                recv_sem=recv_sems.at[slot],
                device_id=(right,),
                device_id_type=pl.DeviceIdType.MESH,
            )
            rdma.start()
            rdma.wait()

            if h < N_DEV - 1:
                c_recv = jnp.mod(my + 2 - h + N_DEV, N_DEV)
                set_chunk(
                    out_ref, c_recv,
                    chunk(out_ref, c_recv)
                    + recv_buf[slot, :, :].astype(jnp.float32),
                )
                send_buf[:, :] = chunk(out_ref, c_recv).astype(jnp.bfloat16)
            else:
                g = h - (N_DEV - 1)
                c_recv = jnp.mod(my + N_DEV - 1 - g, N_DEV)
                set_chunk(
                    out_ref, c_recv, recv_buf[slot, :, :].astype(jnp.float32)
                )
                if h < 2 * (N_DEV - 1) - 1:
                    send_buf[:, :] = recv_buf[slot, :, :]
            if h <= 3:
                pl.semaphore_signal(
                    credit_sem, inc=1,
                    device_id=(left,), device_id_type=pl.DeviceIdType.MESH,
                )

    return pl.pallas_call(
        body,
        out_shape=jax.ShapeDtypeStruct((M, N), jnp.float32),
        in_specs=[
            pl.BlockSpec(memory_space=pltpu.VMEM),
            pl.BlockSpec(memory_space=pltpu.VMEM),
        ],
        out_specs=pl.BlockSpec(memory_space=pltpu.VMEM),
        scratch_shapes=[
            pltpu.VMEM((M_CHUNK, N), jnp.bfloat16),
            pltpu.VMEM((2, M_CHUNK, N), jnp.bfloat16),
            pltpu.SemaphoreType.DMA((2,)),
            pltpu.SemaphoreType.DMA((2,)),
            pltpu.SemaphoreType.REGULAR,
        ],
        compiler_params=pltpu.CompilerParams(
            collective_id=0,
            vmem_limit_bytes=64 * 1024 * 1024,
        ),
    )(x, w)


# device time: 226460 ns/iter; 1.5914x vs baseline; 1.5914x over previous
import jax
import jax.numpy as jnp
from jax import lax
from jax.experimental import pallas as pl
from jax.experimental.pallas import tpu as pltpu

N_DEV = 4
M, K_SHARD, N = 4096, 1024, 2048
M_CHUNK = M // N_DEV
N_HALF = N // 2
N_HOPS = 2 * (N_DEV - 1)


def kernel(x, w_mat):
    x = x.astype(jnp.bfloat16)
    w = w_mat.astype(jnp.bfloat16)

    def body(x_ref, w_ref, out_ref,
             send_r, send_l, recv_r, recv_l,
             ssem_r, rsem_r, ssem_l, rsem_l, credit_r, credit_l):
        my = lax.axis_index("i")
        left = jnp.mod(my + N_DEV - 1, N_DEV)
        right = jnp.mod(my + 1, N_DEV)

        barrier_sem = pltpu.get_barrier_semaphore()
        for nbr in (left, right):
            pl.semaphore_signal(
                barrier_sem, inc=1,
                device_id=(nbr,), device_id_type=pl.DeviceIdType.MESH,
            )
        pl.semaphore_wait(barrier_sem, 2)

        for c in range(N_DEV):
            out_ref[pl.ds(c * M_CHUNK, M_CHUNK), :] = jnp.dot(
                x_ref[pl.ds(c * M_CHUNK, M_CHUNK), :],
                w_ref[:, :],
                preferred_element_type=jnp.float32,
            )

        def half_r(c):
            return out_ref[pl.ds(c * M_CHUNK, M_CHUNK), pl.ds(0, N_HALF)]

        def half_l(c):
            return out_ref[pl.ds(c * M_CHUNK, M_CHUNK), pl.ds(N_HALF, N_HALF)]

        def set_half_r(c, val):
            out_ref[pl.ds(c * M_CHUNK, M_CHUNK), pl.ds(0, N_HALF)] = val

        def set_half_l(c, val):
            out_ref[pl.ds(c * M_CHUNK, M_CHUNK), pl.ds(N_HALF, N_HALF)] = val

        send_r[:, :] = half_r(jnp.mod(my + 3, N_DEV)).astype(jnp.bfloat16)
        send_l[:, :] = half_l(jnp.mod(my + 1, N_DEV)).astype(jnp.bfloat16)

        for h in range(N_HOPS):
            slot = h % 2
            if h >= 2:
                pl.semaphore_wait(credit_r, 1)
                pl.semaphore_wait(credit_l, 1)
            rdma_r = pltpu.make_async_remote_copy(
                src_ref=send_r,
                dst_ref=recv_r.at[slot],
                send_sem=ssem_r.at[slot],
                recv_sem=rsem_r.at[slot],
                device_id=(right,),
                device_id_type=pl.DeviceIdType.MESH,
            )
            rdma_l = pltpu.make_async_remote_copy(
                src_ref=send_l,
                dst_ref=recv_l.at[slot],
                send_sem=ssem_l.at[slot],
                recv_sem=rsem_l.at[slot],
                device_id=(left,),
                device_id_type=pl.DeviceIdType.MESH,
            )
            rdma_r.start()
            rdma_l.start()
            rdma_r.wait()
            rdma_l.wait()

            if h < N_DEV - 1:
                c_r = jnp.mod(my + 2 - h + N_DEV, N_DEV)
                c_l = jnp.mod(my + 2 + h, N_DEV)
                set_half_r(
                    c_r,
                    half_r(c_r) + recv_r[slot, :, :].astype(jnp.float32),
                )
                send_r[:, :] = half_r(c_r).astype(jnp.bfloat16)
                set_half_l(
                    c_l,
                    half_l(c_l) + recv_l[slot, :, :].astype(jnp.float32),
                )
                send_l[:, :] = half_l(c_l).astype(jnp.bfloat16)
            else:
                g = h - (N_DEV - 1)
                c_r = jnp.mod(my + 3 - g + N_DEV, N_DEV)
                c_l = jnp.mod(my + 1 + g, N_DEV)
                if h < N_HOPS - 1:
                    send_r[:, :] = recv_r[slot, :, :]
                    send_l[:, :] = recv_l[slot, :, :]
                set_half_r(c_r, recv_r[slot, :, :].astype(jnp.float32))
                set_half_l(c_l, recv_l[slot, :, :].astype(jnp.float32))
            if h <= 3:
                pl.semaphore_signal(
                    credit_r, inc=1,
                    device_id=(left,), device_id_type=pl.DeviceIdType.MESH,
                )
                pl.semaphore_signal(
                    credit_l, inc=1,
                    device_id=(right,), device_id_type=pl.DeviceIdType.MESH,
                )

    return pl.pallas_call(
        body,
        out_shape=jax.ShapeDtypeStruct((M, N), jnp.float32),
        in_specs=[
            pl.BlockSpec(memory_space=pltpu.VMEM),
            pl.BlockSpec(memory_space=pltpu.VMEM),
        ],
        out_specs=pl.BlockSpec(memory_space=pltpu.VMEM),
        scratch_shapes=[
            pltpu.VMEM((M_CHUNK, N_HALF), jnp.bfloat16),
            pltpu.VMEM((M_CHUNK, N_HALF), jnp.bfloat16),
            pltpu.VMEM((2, M_CHUNK, N_HALF), jnp.bfloat16),
            pltpu.VMEM((2, M_CHUNK, N_HALF), jnp.bfloat16),
            pltpu.SemaphoreType.DMA((2,)),
            pltpu.SemaphoreType.DMA((2,)),
            pltpu.SemaphoreType.DMA((2,)),
            pltpu.SemaphoreType.DMA((2,)),
            pltpu.SemaphoreType.REGULAR,
            pltpu.SemaphoreType.REGULAR,
        ],
        compiler_params=pltpu.CompilerParams(
            collective_id=0,
            vmem_limit_bytes=64 * 1024 * 1024,
        ),
    )(x, w)


# device time: 210816 ns/iter; 1.7095x vs baseline; 1.0742x over previous
import jax
import jax.numpy as jnp
from jax import lax
from jax.experimental import pallas as pl
from jax.experimental.pallas import tpu as pltpu

N_DEV = 4
M, K_SHARD, N = 4096, 1024, 2048
M_CHUNK = M // N_DEV
N_HALF = N // 2
N_HOPS = 2 * (N_DEV - 1)
S = 1
W = N_HALF // S


def kernel(x, w_mat):
    x = x.astype(jnp.bfloat16)
    w = w_mat.astype(jnp.bfloat16)

    def body(x_ref, w_ref, out_ref, recv_r, recv_l, stage_r, stage_l,
             ssem_r, rsem_r, ssem_l, rsem_l, credit_r, credit_l):
        my = lax.axis_index("i")
        left = jnp.mod(my + N_DEV - 1, N_DEV)
        right = jnp.mod(my + 1, N_DEV)

        lanes = []
        for k in range(S):
            lanes.append(dict(d=0, col=k * W, lcol=k * W, to=right,
                              frm=left, recv=recv_r, stage=stage_r,
                              ssem=ssem_r.at[k], rsem=rsem_r.at[k],
                              credit=credit_r.at[k]))
            lanes.append(dict(d=1, col=N_HALF + k * W, lcol=k * W, to=left,
                              frm=right, recv=recv_l, stage=stage_l,
                              ssem=ssem_l.at[k], rsem=rsem_l.at[k],
                              credit=credit_l.at[k]))

        def send_chunk(d, h):
            return jnp.mod(my + 3 - h + 8, N_DEV) if d == 0 else \
                jnp.mod(my + 1 + h, N_DEV)

        def recv_chunk(d, h):
            return jnp.mod(my + 2 - h + 8, N_DEV) if d == 0 else \
                jnp.mod(my + 2 + h, N_DEV)

        def strip(c, col):
            return out_ref[pl.ds(c * M_CHUNK, M_CHUNK), pl.ds(col, W)]

        def set_strip(c, col, val):
            out_ref[pl.ds(c * M_CHUNK, M_CHUNK), pl.ds(col, W)] = val

        def dot_cols(c, col0, ncol, to_ref=None):
            val = jnp.dot(
                x_ref[pl.ds(c * M_CHUNK, M_CHUNK), :],
                w_ref[:, pl.ds(col0, ncol)],
                preferred_element_type=jnp.float32,
            )
            if to_ref is None:
                out_ref[pl.ds(c * M_CHUNK, M_CHUNK), pl.ds(col0, ncol)] = val
            else:
                to_ref[:, :] = val.astype(jnp.bfloat16)

        barrier_sem = pltpu.get_barrier_semaphore()
        for nbr in (left, right):
            pl.semaphore_signal(
                barrier_sem, inc=1,
                device_id=(nbr,), device_id_type=pl.DeviceIdType.MESH,
            )
        pl.semaphore_wait(barrier_sem, 2)

        dot_cols(jnp.mod(my + 3, N_DEV), 0, N_HALF, to_ref=stage_r)
        dot_cols(jnp.mod(my + 1, N_DEV), N_HALF, N_HALF, to_ref=stage_l)

        def issue(ln, h):
            src = (
                ln["stage"].at[:, pl.ds(ln["lcol"], W)] if h == 0
                else ln["recv"].at[(h - 1) % 2, :, pl.ds(ln["lcol"], W)]
            )
            rdma = pltpu.make_async_remote_copy(
                src_ref=src,
                dst_ref=ln["recv"].at[h % 2, :, pl.ds(ln["lcol"], W)],
                send_sem=ln["ssem"].at[h % 2],
                recv_sem=ln["rsem"].at[h % 2],
                device_id=(ln["to"],),
                device_id_type=pl.DeviceIdType.MESH,
            )
            rdma.start()
            return rdma

        pending = {}
        for ln in lanes:
            pending[id(ln)] = issue(ln, 0)

        dot_cols(jnp.mod(my + 2, N_DEV), 0, N)

        for h in range(N_HOPS):
            slot = h % 2
            for ln in lanes:
                d = ln["d"]
                rdma = pending[id(ln)]
                rdma.wait_recv()
                rdma.wait_send()
                if 1 <= h <= 4:
                    pl.semaphore_signal(
                        ln["credit"], inc=1,
                        device_id=(ln["frm"],),
                        device_id_type=pl.DeviceIdType.MESH,
                    )
                c = recv_chunk(d, h)
                if h <= 2:
                    if h == 2:
                        set_strip(
                            c, ln["col"],
                            strip(c, ln["col"])
                            + ln["recv"][slot, :, pl.ds(ln["lcol"], W)]
                            .astype(jnp.float32),
                        )
                        ln["recv"][slot, :, pl.ds(ln["lcol"], W)] = (
                            strip(c, ln["col"]).astype(jnp.bfloat16)
                        )
                    else:
                        ln["recv"][slot, :, pl.ds(ln["lcol"], W)] = (
                            (
                                ln["recv"][slot, :, pl.ds(ln["lcol"], W)]
                                .astype(jnp.float32)
                                + strip(c, ln["col"])
                            ).astype(jnp.bfloat16)
                        )
                if h < N_HOPS - 1:
                    if h + 1 >= 2:
                        pl.semaphore_wait(ln["credit"], 1)
                    pending[id(ln)] = issue(ln, h + 1)
                if h >= 3:
                    set_strip(
                        c, ln["col"],
                        ln["recv"][slot, :, pl.ds(ln["lcol"], W)]
                        .astype(jnp.float32),
                    )
            if h == 0:
                dot_cols(jnp.mod(my + 1, N_DEV), 0, N_HALF)
                dot_cols(jnp.mod(my + 3, N_DEV), N_HALF, N_HALF)
            if h == 1:
                dot_cols(my, 0, N)

    return pl.pallas_call(
        body,
        out_shape=jax.ShapeDtypeStruct((M, N), jnp.float32),
        in_specs=[
            pl.BlockSpec(memory_space=pltpu.VMEM),
            pl.BlockSpec(memory_space=pltpu.VMEM),
        ],
        out_specs=pl.BlockSpec(memory_space=pltpu.VMEM),
        scratch_shapes=[
            pltpu.VMEM((2, M_CHUNK, N_HALF), jnp.bfloat16),
            pltpu.VMEM((2, M_CHUNK, N_HALF), jnp.bfloat16),
            pltpu.VMEM((M_CHUNK, N_HALF), jnp.bfloat16),
            pltpu.VMEM((M_CHUNK, N_HALF), jnp.bfloat16),
            pltpu.SemaphoreType.DMA((S, 2)),
            pltpu.SemaphoreType.DMA((S, 2)),
            pltpu.SemaphoreType.DMA((S, 2)),
            pltpu.SemaphoreType.DMA((S, 2)),
            pltpu.SemaphoreType.REGULAR((S,)),
            pltpu.SemaphoreType.REGULAR((S,)),
        ],
        compiler_params=pltpu.CompilerParams(
            collective_id=0,
            vmem_limit_bytes=64 * 1024 * 1024,
        ),
    )(x, w)


# device time: 198304 ns/iter; 1.8174x vs baseline; 1.0631x over previous
import jax
import jax.numpy as jnp
from jax import lax
from jax.experimental import pallas as pl
from jax.experimental.pallas import tpu as pltpu

N_DEV = 4
M, K_SHARD, N = 4096, 1024, 2048
M_CHUNK = M // N_DEV
N_HALF = N // 2
N_HOPS = 2 * (N_DEV - 1)
S = 2
W = N_HALF // S


def kernel(x, w_mat):
    x = x.astype(jnp.bfloat16)
    w = w_mat.astype(jnp.bfloat16)

    def body(x_ref, w_ref, out_ref, recv_r, recv_l, stage_r, stage_l,
             ssem_r, rsem_r, ssem_l, rsem_l, credit_r, credit_l):
        my = lax.axis_index("i")
        left = jnp.mod(my + N_DEV - 1, N_DEV)
        right = jnp.mod(my + 1, N_DEV)

        lanes = []
        for k in range(S):
            lanes.append(dict(d=0, col=k * W, lcol=k * W, to=right,
                              frm=left, recv=recv_r, stage=stage_r,
                              ssem=ssem_r.at[k], rsem=rsem_r.at[k],
                              credit=credit_r.at[k]))
            lanes.append(dict(d=1, col=N_HALF + k * W, lcol=k * W, to=left,
                              frm=right, recv=recv_l, stage=stage_l,
                              ssem=ssem_l.at[k], rsem=rsem_l.at[k],
                              credit=credit_l.at[k]))

        def send_chunk(d, h):
            return jnp.mod(my + 3 - h + 8, N_DEV) if d == 0 else \
                jnp.mod(my + 1 + h, N_DEV)

        def recv_chunk(d, h):
            return jnp.mod(my + 2 - h + 8, N_DEV) if d == 0 else \
                jnp.mod(my + 2 + h, N_DEV)

        def strip(c, col):
            return out_ref[pl.ds(c * M_CHUNK, M_CHUNK), pl.ds(col, W)]

        def set_strip(c, col, val):
            out_ref[pl.ds(c * M_CHUNK, M_CHUNK), pl.ds(col, W)] = val

        def dot_cols(c, col0, ncol, to_ref=None):
            val = jnp.dot(
                x_ref[pl.ds(c * M_CHUNK, M_CHUNK), :],
                w_ref[:, pl.ds(col0, ncol)],
                preferred_element_type=jnp.float32,
            )
            if to_ref is None:
                out_ref[pl.ds(c * M_CHUNK, M_CHUNK), pl.ds(col0, ncol)] = val
            else:
                to_ref[:, :] = val.astype(jnp.bfloat16)

        barrier_sem = pltpu.get_barrier_semaphore()
        for nbr in (left, right):
            pl.semaphore_signal(
                barrier_sem, inc=1,
                device_id=(nbr,), device_id_type=pl.DeviceIdType.MESH,
            )
        pl.semaphore_wait(barrier_sem, 2)

        dot_cols(jnp.mod(my + 3, N_DEV), 0, N_HALF, to_ref=stage_r)
        dot_cols(jnp.mod(my + 1, N_DEV), N_HALF, N_HALF, to_ref=stage_l)

        def issue(ln, h):
            src = (
                ln["stage"].at[:, pl.ds(ln["lcol"], W)] if h == 0
                else ln["recv"].at[(h - 1) % 2, :, pl.ds(ln["lcol"], W)]
            )
            rdma = pltpu.make_async_remote_copy(
                src_ref=src,
                dst_ref=ln["recv"].at[h % 2, :, pl.ds(ln["lcol"], W)],
                send_sem=ln["ssem"].at[h % 2],
                recv_sem=ln["rsem"].at[h % 2],
                device_id=(ln["to"],),
                device_id_type=pl.DeviceIdType.MESH,
            )
            rdma.start()
            return rdma

        pending = {}
        for ln in lanes:
            pending[id(ln)] = issue(ln, 0)

        dot_cols(jnp.mod(my + 2, N_DEV), 0, N)

        for h in range(N_HOPS):
            slot = h % 2
            for ln in lanes:
                d = ln["d"]
                rdma = pending[id(ln)]
                rdma.wait_recv()
                rdma.wait_send()
                if 1 <= h <= 4:
                    pl.semaphore_signal(
                        ln["credit"], inc=1,
                        device_id=(ln["frm"],),
                        device_id_type=pl.DeviceIdType.MESH,
                    )
                c = recv_chunk(d, h)
                if h <= 2:
                    if h == 2:
                        set_strip(
                            c, ln["col"],
                            strip(c, ln["col"])
                            + ln["recv"][slot, :, pl.ds(ln["lcol"], W)]
                            .astype(jnp.float32),
                        )
                        ln["recv"][slot, :, pl.ds(ln["lcol"], W)] = (
                            strip(c, ln["col"]).astype(jnp.bfloat16)
                        )
                    else:
                        ln["recv"][slot, :, pl.ds(ln["lcol"], W)] = (
                            (
                                ln["recv"][slot, :, pl.ds(ln["lcol"], W)]
                                .astype(jnp.float32)
                                + strip(c, ln["col"])
                            ).astype(jnp.bfloat16)
                        )
                if h < N_HOPS - 1:
                    if h + 1 >= 2:
                        pl.semaphore_wait(ln["credit"], 1)
                    pending[id(ln)] = issue(ln, h + 1)
                if h >= 3:
                    set_strip(
                        c, ln["col"],
                        ln["recv"][slot, :, pl.ds(ln["lcol"], W)]
                        .astype(jnp.float32),
                    )
            if h == 0:
                dot_cols(jnp.mod(my + 1, N_DEV), 0, N_HALF)
                dot_cols(jnp.mod(my + 3, N_DEV), N_HALF, N_HALF)
            if h == 1:
                dot_cols(my, 0, N)

    return pl.pallas_call(
        body,
        out_shape=jax.ShapeDtypeStruct((M, N), jnp.float32),
        in_specs=[
            pl.BlockSpec(memory_space=pltpu.VMEM),
            pl.BlockSpec(memory_space=pltpu.VMEM),
        ],
        out_specs=pl.BlockSpec(memory_space=pltpu.VMEM),
        scratch_shapes=[
            pltpu.VMEM((2, M_CHUNK, N_HALF), jnp.bfloat16),
            pltpu.VMEM((2, M_CHUNK, N_HALF), jnp.bfloat16),
            pltpu.VMEM((M_CHUNK, N_HALF), jnp.bfloat16),
            pltpu.VMEM((M_CHUNK, N_HALF), jnp.bfloat16),
            pltpu.SemaphoreType.DMA((S, 2)),
            pltpu.SemaphoreType.DMA((S, 2)),
            pltpu.SemaphoreType.DMA((S, 2)),
            pltpu.SemaphoreType.DMA((S, 2)),
            pltpu.SemaphoreType.REGULAR((S,)),
            pltpu.SemaphoreType.REGULAR((S,)),
        ],
        compiler_params=pltpu.CompilerParams(
            collective_id=0,
            vmem_limit_bytes=64 * 1024 * 1024,
        ),
    )(x, w)
